# Initial kernel scaffold; baseline (speedup 1.0000x reference)
import jax
import jax.numpy as jnp
from jax.experimental import pallas as pl


def kernel(params, node_feature_name, node_feature_attr, edge_src_n2a, edge_dst_n2a, edge_src_a2n, edge_dst_a2n, edge_label_src, edge_label_dst, node_label_attr):
    raise NotImplementedError("write your pallas kernel here")


# speedup vs baseline: 3.0237x; 3.0237x over previous
import functools

import jax
import jax.numpy as jnp
from jax import lax
from jax.experimental import pallas as pl
from jax.experimental.pallas import tpu as pltpu

H = 256
N_NAME = 10000
N_ATTR = 1000
L = 8192
NEG = 0.01
EPS = 1e-5
BN = 1000
NBLK = N_NAME // BN


def _leaky(x):
    return jnp.where(x >= 0, x, NEG * x)


def _dot(a, b):
    return jnp.dot(a, b, preferred_element_type=jnp.float32)


def _tdot(ct, h):
    return lax.dot_general(ct, h, (((0,), (0,)), ((), ())),
                           preferred_element_type=jnp.float32)


def _enc_name_body(g, w0, b0, w1, b1, ct, x_out, a_out, acc_a):
    i = pl.program_id(0)
    h = _leaky(_dot(g[...], w0[...]) + b0[...])
    h = _leaky(_dot(h, w1[...]) + b1[...])
    x_out[...] = h

    @pl.when(i == 0)
    def _():
        acc_a[...] = jnp.zeros_like(acc_a)

    acc_a[...] += _tdot(ct[...], h)

    @pl.when(i == NBLK - 1)
    def _():
        a_out[...] = acc_a[...]


def _enc_name(g_n, w0, b0, w1, b1, c_n2a_t):
    return pl.pallas_call(
        _enc_name_body,
        grid=(NBLK,),
        in_specs=[
            pl.BlockSpec((BN, H), lambda i: (i, 0)),
            pl.BlockSpec((H, H), lambda i: (0, 0)),
            pl.BlockSpec((1, H), lambda i: (0, 0)),
            pl.BlockSpec((H, H), lambda i: (0, 0)),
            pl.BlockSpec((1, H), lambda i: (0, 0)),
            pl.BlockSpec((BN, N_ATTR), lambda i: (i, 0)),
        ],
        out_specs=[
            pl.BlockSpec((BN, H), lambda i: (i, 0)),
            pl.BlockSpec((N_ATTR, H), lambda i: (0, 0)),
        ],
        out_shape=[
            jax.ShapeDtypeStruct((N_NAME, H), jnp.float32),
            jax.ShapeDtypeStruct((N_ATTR, H), jnp.float32),
        ],
        scratch_shapes=[
            pltpu.VMEM((N_ATTR, H), jnp.float32),
        ],
    )(g_n, w0, b0, w1, b1, c_n2a_t)


def _attr_stage_body(with_enc, with_leaky, *refs):
    if with_enc:
        (g, w0, b0, w1, b1, agg, cnt, ws, wn, bb, gamma, beta, x_enc_out,
         x_out) = refs
        h = _leaky(_dot(g[...], w0[...]) + b0[...])
        h = _leaky(_dot(h, w1[...]) + b1[...])
        x_enc_out[...] = h
    else:
        (g, agg, cnt, ws, wn, bb, gamma, beta, x_out) = refs
        h = g[...]
    aggr = agg[...] / jnp.maximum(cnt[...], 1.0)
    pre = _dot(h, ws[...]) + _dot(aggr, wn[...]) + bb[...]
    mu = jnp.mean(pre, axis=0, keepdims=True)
    var = jnp.mean((pre - mu) ** 2, axis=0, keepdims=True)
    y = (pre - mu) * lax.rsqrt(var + EPS) * gamma[...] + beta[...]
    if with_leaky:
        y = _leaky(y)
    x_out[...] = y


def _attr_stage(with_enc, with_leaky, args):
    n_in = len(args)
    n_out = 2 if with_enc else 1
    full = lambda s: pl.BlockSpec(s, lambda: (0, 0))
    in_specs = [full(a.shape) for a in args]
    return pl.pallas_call(
        functools.partial(_attr_stage_body, with_enc, with_leaky),
        grid=(),
        in_specs=in_specs,
        out_specs=[full((N_ATTR, H))] * n_out,
        out_shape=[jax.ShapeDtypeStruct((N_ATTR, H), jnp.float32)] * n_out,
    )(*args)


def _name_conv_body(x, c, xa, ws, wn, bb, pre_out, stats_out, s1, s2):
    i = pl.program_id(0)
    cb = c[...]
    rs = jnp.sum(cb, axis=1, keepdims=True)
    aggr = _dot(cb, xa[...]) / jnp.maximum(rs, 1.0)
    pre = _dot(x[...], ws[...]) + _dot(aggr, wn[...]) + bb[...]
    pre_out[...] = pre

    @pl.when(i == 0)
    def _():
        s1[...] = jnp.zeros_like(s1)
        s2[...] = jnp.zeros_like(s2)

    s1[...] += jnp.sum(pre, axis=0, keepdims=True)
    s2[...] += jnp.sum(pre * pre, axis=0, keepdims=True)

    @pl.when(i == NBLK - 1)
    def _():
        stats_out[0:1, :] = s1[...]
        stats_out[1:2, :] = s2[...]


def _name_conv(x_n, c_a2n, x_a, ws, wn, bb):
    return pl.pallas_call(
        _name_conv_body,
        grid=(NBLK,),
        in_specs=[
            pl.BlockSpec((BN, H), lambda i: (i, 0)),
            pl.BlockSpec((BN, N_ATTR), lambda i: (i, 0)),
            pl.BlockSpec((N_ATTR, H), lambda i: (0, 0)),
            pl.BlockSpec((H, H), lambda i: (0, 0)),
            pl.BlockSpec((H, H), lambda i: (0, 0)),
            pl.BlockSpec((1, H), lambda i: (0, 0)),
        ],
        out_specs=[
            pl.BlockSpec((BN, H), lambda i: (i, 0)),
            pl.BlockSpec((2, H), lambda i: (0, 0)),
        ],
        out_shape=[
            jax.ShapeDtypeStruct((N_NAME, H), jnp.float32),
            jax.ShapeDtypeStruct((2, H), jnp.float32),
        ],
        scratch_shapes=[
            pltpu.VMEM((1, H), jnp.float32),
            pltpu.VMEM((1, H), jnp.float32),
        ],
    )(x_n, c_a2n, x_a, ws, wn, bb)


def _bn_accum_body(pre, stats, gamma, beta, c, x_out, a_out, acc):
    i = pl.program_id(0)
    mu = stats[0:1, :] / N_NAME
    var = stats[1:2, :] / N_NAME - mu * mu
    y = (pre[...] - mu) * lax.rsqrt(var + EPS) * gamma[...] + beta[...]
    y = _leaky(y)
    x_out[...] = y

    @pl.when(i == 0)
    def _():
        acc[...] = jnp.zeros_like(acc)

    acc[...] += _tdot(c[...], y)

    @pl.when(i == NBLK - 1)
    def _():
        a_out[...] = acc[...]


def _bn_accum(pre_n, stats, gamma, beta, c_n2a_t):
    return pl.pallas_call(
        _bn_accum_body,
        grid=(NBLK,),
        in_specs=[
            pl.BlockSpec((BN, H), lambda i: (i, 0)),
            pl.BlockSpec((2, H), lambda i: (0, 0)),
            pl.BlockSpec((1, H), lambda i: (0, 0)),
            pl.BlockSpec((1, H), lambda i: (0, 0)),
            pl.BlockSpec((BN, N_ATTR), lambda i: (i, 0)),
        ],
        out_specs=[
            pl.BlockSpec((BN, H), lambda i: (i, 0)),
            pl.BlockSpec((N_ATTR, H), lambda i: (0, 0)),
        ],
        out_shape=[
            jax.ShapeDtypeStruct((N_NAME, H), jnp.float32),
            jax.ShapeDtypeStruct((N_ATTR, H), jnp.float32),
        ],
        scratch_shapes=[pltpu.VMEM((N_ATTR, H), jnp.float32)],
    )(pre_n, stats, gamma, beta, c_n2a_t)


LB = 1024
LBLK = L // LB


def _distmult_body(rows, stats, gamma, beta, xa, out):
    mu = stats[0:1, :] / N_NAME
    var = stats[1:2, :] / N_NAME - mu * mu
    y = (rows[...] - mu) * lax.rsqrt(var + EPS) * gamma[...] + beta[...]
    out[...] = lax.dot_general(y, xa[...], (((1,), (1,)), ((), ())),
                               preferred_element_type=jnp.float32)


def _distmult(rows, stats, gamma, beta, x_a):
    return pl.pallas_call(
        _distmult_body,
        grid=(LBLK,),
        in_specs=[
            pl.BlockSpec((LB, H), lambda i: (i, 0)),
            pl.BlockSpec((2, H), lambda i: (0, 0)),
            pl.BlockSpec((1, H), lambda i: (0, 0)),
            pl.BlockSpec((1, H), lambda i: (0, 0)),
            pl.BlockSpec((N_ATTR, H), lambda i: (0, 0)),
        ],
        out_specs=pl.BlockSpec((LB, N_ATTR), lambda i: (i, 0)),
        out_shape=jax.ShapeDtypeStruct((L, N_ATTR), jnp.float32),
    )(rows, stats, gamma, beta, x_a)


def _tile_body(lbl, out):
    out[...] = jnp.broadcast_to(lbl[...], (LB, N_ATTR))


def _tile_labels(lbl_row):
    return pl.pallas_call(
        _tile_body,
        grid=(LBLK,),
        in_specs=[pl.BlockSpec((1, N_ATTR), lambda i: (0, 0))],
        out_specs=pl.BlockSpec((LB, N_ATTR), lambda i: (i, 0)),
        out_shape=jax.ShapeDtypeStruct((L, N_ATTR), jnp.int32),
    )(lbl_row)


def kernel(params, node_feature_name, node_feature_attr, edge_src_n2a,
           edge_dst_n2a, edge_src_a2n, edge_dst_a2n, edge_label_src,
           edge_label_dst, node_label_attr):
    p = params
    r = lambda v: jnp.reshape(v, (1, H))

    g_n = jnp.take(p['emb_name'], node_feature_name[:, 0], axis=0)
    g_a = jnp.take(p['emb_attr'], node_feature_attr[:, 0], axis=0)
    c_a2n = jnp.zeros((N_NAME, N_ATTR), jnp.float32).at[
        edge_dst_a2n, edge_src_a2n].add(1.0)
    c_n2a_t = jnp.zeros((N_NAME, N_ATTR), jnp.float32).at[
        edge_src_n2a, edge_dst_n2a].add(1.0)
    cnt_a = jnp.zeros((N_ATTR, 1), jnp.float32).at[edge_dst_n2a, 0].add(1.0)

    x0_n, a0_a = _enc_name(
        g_n, p['mlp_name_W0'], r(p['mlp_name_b0']),
        p['mlp_name_W1'], r(p['mlp_name_b1']), c_n2a_t)
    x0_a, x1_a = _attr_stage(True, True, (
        g_a, p['mlp_attr_W0'], r(p['mlp_attr_b0']),
        p['mlp_attr_W1'], r(p['mlp_attr_b1']),
        a0_a, cnt_a,
        p['conv0_n2a_Wself'], p['conv0_n2a_Wneigh'], r(p['conv0_n2a_b']),
        r(p['bn0_attr_gamma']), r(p['bn0_attr_beta'])))
    pre_n1, stats1 = _name_conv(
        x0_n, c_a2n, x0_a,
        p['conv0_a2n_Wself'], p['conv0_a2n_Wneigh'], r(p['conv0_a2n_b']))
    x1_n, a1_a = _bn_accum(pre_n1, stats1, r(p['bn0_name_gamma']),
                           r(p['bn0_name_beta']), c_n2a_t)
    (x2_a,) = _attr_stage(False, False, (
        x1_a, a1_a, cnt_a,
        p['conv1_n2a_Wself'], p['conv1_n2a_Wneigh'], r(p['conv1_n2a_b']),
        r(p['bn1_attr_gamma']), r(p['bn1_attr_beta'])))
    pre_n2, stats2 = _name_conv(
        x1_n, c_a2n, x1_a,
        p['conv1_a2n_Wself'], p['conv1_a2n_Wneigh'], r(p['conv1_a2n_b']))

    rows = jnp.take(pre_n2, edge_label_src, axis=0)
    out = _distmult(rows, stats2, r(p['bn1_name_gamma']),
                    r(p['bn1_name_beta']), x2_a)

    pred = _tile_labels(jnp.reshape(node_label_attr, (1, N_ATTR)))
    return (out, edge_label_dst, jnp.reshape(pred, (L * N_ATTR,)))
